# baseline (speedup 1.0000x reference)
import math

import jax
import jax.numpy as jnp
import numpy as np
from jax.experimental import pallas as pl
from jax.experimental.pallas import tpu as pltpu
from jax.experimental.shard_map import shard_map

L = 10
H = L * L
B = 16384
S = B // 128


def _deint_mats():
    e = np.zeros((2, 128, 256), dtype=np.float32)
    for k in range(64):
        e[0, 2 * k, k] = 1.0
        e[1, 2 * k, 64 + k] = 1.0
        e[0, 2 * k + 1, 128 + k] = 1.0
        e[1, 2 * k + 1, 192 + k] = 1.0
    return e


_DEINT_NP = _deint_mats()


def _sh_block(flat_ref, e_ref, out_ref):
    f = flat_ref[0]
    f3 = f.reshape(S, 2, 128)
    f0 = f3[:, 0, :]
    f1 = f3[:, 1, :]
    p = (jnp.dot(f0, e_ref[0], preferred_element_type=jnp.float32,
                 precision=jax.lax.Precision.HIGHEST)
         + jnp.dot(f1, e_ref[1], preferred_element_type=jnp.float32,
                   precision=jax.lax.Precision.HIGHEST))
    lon = p[:, :128]
    lat = p[:, 128:]
    rad = math.pi / 180.0
    phi = (lon + 180.0) * rad
    theta = (lat + 90.0) * rad
    x = jnp.cos(theta)
    sx = jnp.sin(theta)

    c = [jnp.ones_like(phi), jnp.cos(phi)]
    s = [jnp.zeros_like(phi), jnp.sin(phi)]
    two_c1 = 2.0 * c[1]
    for m in range(2, L):
        c.append(two_c1 * c[m - 1] - c[m - 2])
        s.append(two_c1 * s[m - 1] - s[m - 2])

    P = {(0, 0): jnp.ones_like(x)}
    for m in range(1, L):
        df = 1.0
        for k in range(1, 2 * m, 2):
            df *= float(k)
        P[(m, m)] = ((-1.0) ** m) * df * (sx ** m)
    for m in range(0, L - 1):
        P[(m + 1, m)] = (2.0 * m + 1.0) * x * P[(m, m)]
    for m in range(0, L):
        for l in range(m + 2, L):
            a = (2.0 * l - 1.0) / float(l - m)
            b = (l + m - 1.0) / float(l - m)
            P[(l, m)] = a * x * P[(l - 1, m)] - b * P[(l - 2, m)]

    ys = []
    for l in range(L):
        for m in range(-l, l + 1):
            am = abs(m)
            K = math.sqrt((2.0 * l + 1.0) / (4.0 * math.pi)
                          * math.factorial(l - am) / math.factorial(l + am))
            if m > 0:
                ys.append((math.sqrt(2.0) * K) * (c[m] * P[(l, am)]))
            elif m < 0:
                ys.append((math.sqrt(2.0) * K) * (s[am] * P[(l, am)]))
            else:
                ys.append(K * P[(l, 0)])

    yt = jnp.stack(ys, axis=0).reshape(H, B)
    out_ref[...] = yt.T


def _sh_pallas(lonlat):
    n = lonlat.shape[0]
    g = -(-n // B)
    npad = g * B
    flat = jnp.pad(lonlat.reshape(-1), (0, (npad - n) * 2))
    flat = flat.reshape(g, 2 * S, 128)
    return pl.pallas_call(
        _sh_block,
        grid=(g,),
        in_specs=[
            pl.BlockSpec((1, 2 * S, 128), lambda i: (i, 0, 0)),
            pl.BlockSpec((2, 128, 256), lambda i: (0, 0, 0)),
        ],
        out_specs=pl.BlockSpec((B, H), lambda i: (i, 0)),
        out_shape=jax.ShapeDtypeStruct((n, H), jnp.float32),
        compiler_params=pltpu.CompilerParams(
            dimension_semantics=("parallel",),
        ),
    )(flat, jnp.asarray(_DEINT_NP))


def kernel(lonlat):
    return _sh_pallas(lonlat)


# speedup vs baseline: 6.7006x; 6.7006x over previous
import math

import jax
import jax.numpy as jnp
import numpy as np
from jax.experimental import pallas as pl
from jax.experimental.pallas import tpu as pltpu
from jax.experimental.shard_map import shard_map

L = 10
H = L * L
B = 16384
S = B // 128


def _sh_block(pl_ref, out_ref):
    lon = pl_ref[0, 0]
    lat = pl_ref[1, 0]
    rad = math.pi / 180.0
    phi = (lon + 180.0) * rad
    theta = (lat + 90.0) * rad
    x = jnp.cos(theta)
    sx = jnp.sin(theta)

    c = [jnp.ones_like(phi), jnp.cos(phi)]
    s = [jnp.zeros_like(phi), jnp.sin(phi)]
    two_c1 = 2.0 * c[1]
    for m in range(2, L):
        c.append(two_c1 * c[m - 1] - c[m - 2])
        s.append(two_c1 * s[m - 1] - s[m - 2])

    P = {(0, 0): jnp.ones_like(x)}
    for m in range(1, L):
        df = 1.0
        for k in range(1, 2 * m, 2):
            df *= float(k)
        P[(m, m)] = ((-1.0) ** m) * df * (sx ** m)
    for m in range(0, L - 1):
        P[(m + 1, m)] = (2.0 * m + 1.0) * x * P[(m, m)]
    for m in range(0, L):
        for l in range(m + 2, L):
            a = (2.0 * l - 1.0) / float(l - m)
            b = (l + m - 1.0) / float(l - m)
            P[(l, m)] = a * x * P[(l - 1, m)] - b * P[(l - 2, m)]

    ys = []
    for l in range(L):
        for m in range(-l, l + 1):
            am = abs(m)
            K = math.sqrt((2.0 * l + 1.0) / (4.0 * math.pi)
                          * math.factorial(l - am) / math.factorial(l + am))
            if m > 0:
                ys.append((math.sqrt(2.0) * K) * (c[m] * P[(l, am)]))
            elif m < 0:
                ys.append((math.sqrt(2.0) * K) * (s[am] * P[(l, am)]))
            else:
                ys.append(K * P[(l, 0)])

    out_ref[...] = jnp.stack(ys, axis=0).reshape(H, B)


def _sh_pallas(lonlat):
    n = lonlat.shape[0]
    g = -(-n // B)
    npad = g * B
    lonT = jnp.pad(lonlat.T, ((0, 0), (0, npad - n)))
    planes = lonT.reshape(2, g, S, 128)
    out_t = pl.pallas_call(
        _sh_block,
        grid=(g,),
        in_specs=[
            pl.BlockSpec((2, 1, S, 128), lambda i: (0, i, 0, 0)),
        ],
        out_specs=pl.BlockSpec((H, B), lambda i: (0, i)),
        out_shape=jax.ShapeDtypeStruct((H, n), jnp.float32),
        compiler_params=pltpu.CompilerParams(
            dimension_semantics=("parallel",),
        ),
    )(planes)
    return out_t.T


def kernel(lonlat):
    return _sh_pallas(lonlat)


# speedup vs baseline: 6.7722x; 1.0107x over previous
import math

import jax
import jax.numpy as jnp
import numpy as np
from jax.experimental import pallas as pl
from jax.experimental.pallas import tpu as pltpu
from jax.experimental.shard_map import shard_map

L = 10
H = L * L
B = 32768
S = B // 128


def _sh_block(pl_ref, out_ref):
    lon = pl_ref[0, 0]
    lat = pl_ref[1, 0]
    rad = math.pi / 180.0
    phi = (lon + 180.0) * rad
    theta = (lat + 90.0) * rad
    x = jnp.cos(theta)
    sx = jnp.sin(theta)

    c = [jnp.ones_like(phi), jnp.cos(phi)]
    s = [jnp.zeros_like(phi), jnp.sin(phi)]
    two_c1 = 2.0 * c[1]
    for m in range(2, L):
        c.append(two_c1 * c[m - 1] - c[m - 2])
        s.append(two_c1 * s[m - 1] - s[m - 2])

    P = {(0, 0): jnp.ones_like(x)}
    for m in range(1, L):
        df = 1.0
        for k in range(1, 2 * m, 2):
            df *= float(k)
        P[(m, m)] = ((-1.0) ** m) * df * (sx ** m)
    for m in range(0, L - 1):
        P[(m + 1, m)] = (2.0 * m + 1.0) * x * P[(m, m)]
    for m in range(0, L):
        for l in range(m + 2, L):
            a = (2.0 * l - 1.0) / float(l - m)
            b = (l + m - 1.0) / float(l - m)
            P[(l, m)] = a * x * P[(l - 1, m)] - b * P[(l - 2, m)]

    ys = []
    for l in range(L):
        for m in range(-l, l + 1):
            am = abs(m)
            K = math.sqrt((2.0 * l + 1.0) / (4.0 * math.pi)
                          * math.factorial(l - am) / math.factorial(l + am))
            if m > 0:
                ys.append((math.sqrt(2.0) * K) * (c[m] * P[(l, am)]))
            elif m < 0:
                ys.append((math.sqrt(2.0) * K) * (s[am] * P[(l, am)]))
            else:
                ys.append(K * P[(l, 0)])

    out_ref[...] = jnp.stack(ys, axis=0).reshape(H, B)


def _sh_pallas(lonlat):
    n = lonlat.shape[0]
    g = -(-n // B)
    npad = g * B
    lonT = jnp.pad(lonlat.T, ((0, 0), (0, npad - n)))
    planes = lonT.reshape(2, g, S, 128)
    out_t = pl.pallas_call(
        _sh_block,
        grid=(g,),
        in_specs=[
            pl.BlockSpec((2, 1, S, 128), lambda i: (0, i, 0, 0)),
        ],
        out_specs=pl.BlockSpec((H, B), lambda i: (0, i)),
        out_shape=jax.ShapeDtypeStruct((H, n), jnp.float32),
        compiler_params=pltpu.CompilerParams(
            dimension_semantics=("parallel",),
        ),
    )(planes)
    return out_t.T


def kernel(lonlat):
    return _sh_pallas(lonlat)


# speedup vs baseline: 7.3461x; 1.0847x over previous
import math

import jax
import jax.numpy as jnp
import numpy as np
from jax.experimental import pallas as pl
from jax.experimental.pallas import tpu as pltpu
from jax.experimental.shard_map import shard_map

L = 10
H = L * L
B = 32768
S = B // 128


def _sh_block(pl_ref, out_ref):
    lon = pl_ref[0:1, :].reshape(S, 128)
    lat = pl_ref[1:2, :].reshape(S, 128)
    rad = math.pi / 180.0
    phi = (lon + 180.0) * rad
    theta = (lat + 90.0) * rad
    x = jnp.cos(theta)
    sx = jnp.sin(theta)

    c = [jnp.ones_like(phi), jnp.cos(phi)]
    s = [jnp.zeros_like(phi), jnp.sin(phi)]
    two_c1 = 2.0 * c[1]
    for m in range(2, L):
        c.append(two_c1 * c[m - 1] - c[m - 2])
        s.append(two_c1 * s[m - 1] - s[m - 2])

    P = {(0, 0): jnp.ones_like(x)}
    for m in range(1, L):
        df = 1.0
        for k in range(1, 2 * m, 2):
            df *= float(k)
        P[(m, m)] = ((-1.0) ** m) * df * (sx ** m)
    for m in range(0, L - 1):
        P[(m + 1, m)] = (2.0 * m + 1.0) * x * P[(m, m)]
    for m in range(0, L):
        for l in range(m + 2, L):
            a = (2.0 * l - 1.0) / float(l - m)
            b = (l + m - 1.0) / float(l - m)
            P[(l, m)] = a * x * P[(l - 1, m)] - b * P[(l - 2, m)]

    ys = []
    for l in range(L):
        for m in range(-l, l + 1):
            am = abs(m)
            K = math.sqrt((2.0 * l + 1.0) / (4.0 * math.pi)
                          * math.factorial(l - am) / math.factorial(l + am))
            if m > 0:
                ys.append((math.sqrt(2.0) * K) * (c[m] * P[(l, am)]))
            elif m < 0:
                ys.append((math.sqrt(2.0) * K) * (s[am] * P[(l, am)]))
            else:
                ys.append(K * P[(l, 0)])

    out_ref[...] = jnp.stack(ys, axis=0).reshape(H, B)


def _sh_pallas(lonlat):
    n = lonlat.shape[0]
    g = -(-n // B)
    out_t = pl.pallas_call(
        _sh_block,
        grid=(g,),
        in_specs=[
            pl.BlockSpec((2, B), lambda i: (0, i)),
        ],
        out_specs=pl.BlockSpec((H, B), lambda i: (0, i)),
        out_shape=jax.ShapeDtypeStruct((H, n), jnp.float32),
        compiler_params=pltpu.CompilerParams(
            dimension_semantics=("parallel",),
        ),
    )(lonlat.T)
    return out_t.T


def kernel(lonlat):
    return _sh_pallas(lonlat)
